# baseline (speedup 1.0000x reference)
import functools

import jax
import jax.numpy as jnp
from jax import lax
from jax.experimental import pallas as pl
from jax.experimental.pallas import tpu as pltpu
from jax.experimental.pallas import tpu_sc as plsc

_NC = 2
_NS = 16
_NW = _NC * _NS
_C = 4
_L = 16


def _make_sc_kernel(n, k_nb, f_feat, chunks_per_worker):
    nf = f_feat // _L
    inv_k = 1.0 / float(k_nb)
    cw = _C * k_nb
    rpt = ((n + _NS * 8 - 1) // (_NS * 8)) * 8
    tail = n - (_NS - 1) * rpt
    assert 0 < tail <= rpt and tail % 8 == 0
    n_chunks = n // _C
    mesh = plsc.VectorSubcoreMesh(core_axis_name="c", subcore_axis_name="s")

    @functools.partial(
        pl.kernel,
        mesh=mesh,
        out_type=jax.ShapeDtypeStruct((n, 2 * f_feat), jnp.float32),
        scratch_types=[
            pltpu.VMEM_SHARED((n, f_feat), jnp.float32),
            pltpu.VMEM((chunks_per_worker * cw,), jnp.int32),
            pltpu.VMEM((cw, f_feat), jnp.float32),
            pltpu.VMEM((cw, f_feat), jnp.float32),
            pltpu.VMEM((_C, 2 * f_feat), jnp.float32),
            pltpu.VMEM((_C, 2 * f_feat), jnp.float32),
            pltpu.SemaphoreType.DMA,
            pltpu.SemaphoreType.DMA,
            pltpu.SemaphoreType.DMA,
            pltpu.SemaphoreType.DMA,
        ],
    )
    def sc_kernel(x_hbm, idxs_hbm, out_hbm, xs_shared, idx_all, rows0, rows1,
                  out_v0, out_v1, sem_g0, sem_g1, sem_o0, sem_o1):
        sid = lax.axis_index("s")
        wid = sid * _NC + lax.axis_index("c")
        chunk0 = jnp.minimum(wid * chunks_per_worker,
                             n_chunks - chunks_per_worker)

        @pl.when(sid < _NS - 1)
        def _():
            pltpu.sync_copy(x_hbm.at[pl.ds(sid * rpt, rpt)],
                            xs_shared.at[pl.ds(sid * rpt, rpt)])

        @pl.when(sid == _NS - 1)
        def _():
            pltpu.sync_copy(x_hbm.at[pl.ds((_NS - 1) * rpt, tail)],
                            xs_shared.at[pl.ds((_NS - 1) * rpt, tail)])

        pltpu.sync_copy(
            idxs_hbm.at[pl.ds(chunk0 * cw, chunks_per_worker * cw)], idx_all)
        plsc.subcore_barrier()

        def start_gather(ci, rows, sem):
            pltpu.async_copy(xs_shared.at[idx_all.at[pl.ds(ci * cw, cw)]],
                             rows, sem)

        def compute_chunk(ci, rows, out_v, sem_o):
            zero = jnp.zeros((_L,), jnp.float32)
            ninf = jnp.full((_L,), -jnp.inf, jnp.float32)
            for v in range(_C):
                r0 = v * k_nb

                def k_body(kq, acc, r0=r0):
                    sums, maxs = acc
                    for dk in range(8):
                        r = r0 + kq * 8 + dk
                        vals = [rows[r, pl.ds(f * _L, _L)] for f in range(nf)]
                        sums = tuple(s + x for s, x in zip(sums, vals))
                        maxs = tuple(jnp.maximum(m, x)
                                     for m, x in zip(maxs, vals))
                    return sums, maxs

                sums, maxs = lax.fori_loop(
                    0, k_nb // 8, k_body, ((zero,) * nf, (ninf,) * nf))
                for f in range(nf):
                    out_v[v, pl.ds(f * _L, _L)] = sums[f] * inv_k
                    out_v[v, pl.ds(f_feat + f * _L, _L)] = maxs[f]
            base = (chunk0 + ci) * _C
            pltpu.async_copy(out_v, out_hbm.at[pl.ds(base, _C)], sem_o)

        start_gather(0, rows0, sem_g0)

        def pair_body(i, carry):
            ci0 = i * 2
            start_gather(ci0 + 1, rows1, sem_g1)
            pltpu.make_async_copy(
                xs_shared.at[idx_all.at[pl.ds(0, cw)]], rows0, sem_g0).wait()

            @pl.when(i > 0)
            def _():
                pltpu.make_async_copy(out_v0, out_hbm.at[pl.ds(0, _C)],
                                      sem_o0).wait()
            compute_chunk(ci0, rows0, out_v0, sem_o0)

            @pl.when(ci0 + 2 < chunks_per_worker)
            def _():
                start_gather(ci0 + 2, rows0, sem_g0)
            pltpu.make_async_copy(
                xs_shared.at[idx_all.at[pl.ds(0, cw)]], rows1, sem_g1).wait()

            @pl.when(i > 0)
            def _():
                pltpu.make_async_copy(out_v1, out_hbm.at[pl.ds(0, _C)],
                                      sem_o1).wait()
            compute_chunk(ci0 + 1, rows1, out_v1, sem_o1)
            return carry

        lax.fori_loop(0, chunks_per_worker // 2, pair_body, 0)

        pltpu.make_async_copy(out_v0, out_hbm.at[pl.ds(0, _C)], sem_o0).wait()
        pltpu.make_async_copy(out_v1, out_hbm.at[pl.ds(0, _C)], sem_o1).wait()

    return sc_kernel


def kernel(x, idxs):
    n, f_feat = x.shape
    k_nb = idxs.shape[1]
    assert k_nb % 4 == 0 and f_feat % _L == 0 and n % _C == 0
    n_chunks = n // _C
    chunks_per_worker = ((n_chunks + 2 * _NW - 1) // (2 * _NW)) * 2
    assert n_chunks >= chunks_per_worker

    sc_kernel = _make_sc_kernel(n, k_nb, f_feat, chunks_per_worker)
    return sc_kernel(x, idxs.reshape(-1))


# speedup vs baseline: 1.0162x; 1.0162x over previous
import functools

import jax
import jax.numpy as jnp
from jax import lax
from jax.experimental import pallas as pl
from jax.experimental.pallas import tpu as pltpu
from jax.experimental.pallas import tpu_sc as plsc

_NC = 2
_NS = 16
_NW = _NC * _NS
_C = 4
_L = 16


def _make_sc_kernel(n, k_nb, f_feat, chunks_per_worker):
    nf = f_feat // _L
    inv_k = 1.0 / float(k_nb)
    cw = _C * k_nb
    rpt = ((n + _NS * 8 - 1) // (_NS * 8)) * 8
    tail = n - (_NS - 1) * rpt
    assert 0 < tail <= rpt and tail % 8 == 0
    n_chunks = n // _C
    mesh = plsc.VectorSubcoreMesh(core_axis_name="c", subcore_axis_name="s")

    @functools.partial(
        pl.kernel,
        mesh=mesh,
        out_type=jax.ShapeDtypeStruct((n, 2 * f_feat), jnp.float32),
        scratch_types=[
            pltpu.VMEM_SHARED((n, f_feat), jnp.float32),
            pltpu.VMEM((chunks_per_worker * cw,), jnp.int32),
            pltpu.VMEM((cw, f_feat), jnp.float32),
            pltpu.VMEM((cw, f_feat), jnp.float32),
            pltpu.VMEM((_C, 2 * f_feat), jnp.float32),
            pltpu.VMEM((_C, 2 * f_feat), jnp.float32),
            pltpu.SemaphoreType.DMA,
            pltpu.SemaphoreType.DMA,
            pltpu.SemaphoreType.DMA,
            pltpu.SemaphoreType.DMA,
        ],
    )
    def sc_kernel(x_hbm, idxs_hbm, out_hbm, xs_shared, idx_all, rows0, rows1,
                  out_v0, out_v1, sem_g0, sem_g1, sem_o0, sem_o1):
        sid = lax.axis_index("s")
        wid = sid * _NC + lax.axis_index("c")
        chunk0 = jnp.minimum(wid * chunks_per_worker,
                             n_chunks - chunks_per_worker)

        @pl.when(sid < _NS - 1)
        def _():
            pltpu.sync_copy(x_hbm.at[pl.ds(sid * rpt, rpt)],
                            xs_shared.at[pl.ds(sid * rpt, rpt)])

        @pl.when(sid == _NS - 1)
        def _():
            pltpu.sync_copy(x_hbm.at[pl.ds((_NS - 1) * rpt, tail)],
                            xs_shared.at[pl.ds((_NS - 1) * rpt, tail)])

        pltpu.sync_copy(
            idxs_hbm.at[pl.ds(chunk0 * cw, chunks_per_worker * cw)], idx_all)
        plsc.subcore_barrier()

        def start_gather(ci, rows, sem):
            pltpu.async_copy(xs_shared.at[idx_all.at[pl.ds(ci * cw, cw)]],
                             rows, sem)

        def compute_chunk(ci, rows, out_v, sem_o):
            zero = jnp.zeros((_L,), jnp.float32)
            ninf = jnp.full((_L,), -jnp.inf, jnp.float32)
            for v in range(_C):
                r0 = v * k_nb

                def k_body(kq, acc, r0=r0):
                    sums, maxs = acc
                    for dk in range(4):
                        r = r0 + kq * 4 + dk
                        vals = [rows[r, pl.ds(f * _L, _L)] for f in range(nf)]
                        sums = tuple(s + x for s, x in zip(sums, vals))
                        maxs = tuple(jnp.maximum(m, x)
                                     for m, x in zip(maxs, vals))
                    return sums, maxs

                sums, maxs = lax.fori_loop(
                    0, k_nb // 4, k_body, ((zero,) * nf, (ninf,) * nf))
                for f in range(nf):
                    out_v[v, pl.ds(f * _L, _L)] = sums[f] * inv_k
                    out_v[v, pl.ds(f_feat + f * _L, _L)] = maxs[f]
            base = (chunk0 + ci) * _C
            pltpu.async_copy(out_v, out_hbm.at[pl.ds(base, _C)], sem_o)

        start_gather(0, rows0, sem_g0)

        def pair_body(i, carry):
            ci0 = i * 2
            start_gather(ci0 + 1, rows1, sem_g1)
            pltpu.make_async_copy(
                xs_shared.at[idx_all.at[pl.ds(0, cw)]], rows0, sem_g0).wait()

            @pl.when(i > 0)
            def _():
                pltpu.make_async_copy(out_v0, out_hbm.at[pl.ds(0, _C)],
                                      sem_o0).wait()
            compute_chunk(ci0, rows0, out_v0, sem_o0)

            @pl.when(ci0 + 2 < chunks_per_worker)
            def _():
                start_gather(ci0 + 2, rows0, sem_g0)
            pltpu.make_async_copy(
                xs_shared.at[idx_all.at[pl.ds(0, cw)]], rows1, sem_g1).wait()

            @pl.when(i > 0)
            def _():
                pltpu.make_async_copy(out_v1, out_hbm.at[pl.ds(0, _C)],
                                      sem_o1).wait()
            compute_chunk(ci0 + 1, rows1, out_v1, sem_o1)
            return carry

        lax.fori_loop(0, chunks_per_worker // 2, pair_body, 0)

        pltpu.make_async_copy(out_v0, out_hbm.at[pl.ds(0, _C)], sem_o0).wait()
        pltpu.make_async_copy(out_v1, out_hbm.at[pl.ds(0, _C)], sem_o1).wait()

    return sc_kernel


def kernel(x, idxs):
    n, f_feat = x.shape
    k_nb = idxs.shape[1]
    assert k_nb % 4 == 0 and f_feat % _L == 0 and n % _C == 0
    n_chunks = n // _C
    chunks_per_worker = ((n_chunks + 2 * _NW - 1) // (2 * _NW)) * 2
    assert n_chunks >= chunks_per_worker

    sc_kernel = _make_sc_kernel(n, k_nb, f_feat, chunks_per_worker)
    return sc_kernel(x, idxs.reshape(-1))


# speedup vs baseline: 1.0262x; 1.0099x over previous
import functools

import jax
import jax.numpy as jnp
from jax import lax
from jax.experimental import pallas as pl
from jax.experimental.pallas import tpu as pltpu
from jax.experimental.pallas import tpu_sc as plsc

_NC = 2
_NS = 16
_NW = _NC * _NS
_C = 4
_L = 16


def _make_sc_kernel(n, k_nb, f_feat, chunks_per_worker):
    nf = f_feat // _L
    inv_k = 1.0 / float(k_nb)
    cw = _C * k_nb
    rpt = ((n + _NS * 8 - 1) // (_NS * 8)) * 8
    tail = n - (_NS - 1) * rpt
    assert 0 < tail <= rpt and tail % 8 == 0
    n_chunks = n // _C
    mesh = plsc.VectorSubcoreMesh(core_axis_name="c", subcore_axis_name="s")

    @functools.partial(
        pl.kernel,
        mesh=mesh,
        out_type=jax.ShapeDtypeStruct((n, 2 * f_feat), jnp.float32),
        scratch_types=[
            pltpu.VMEM_SHARED((n, f_feat), jnp.float32),
            pltpu.VMEM((chunks_per_worker * cw,), jnp.int32),
            pltpu.VMEM((cw, f_feat), jnp.float32),
            pltpu.VMEM((cw, f_feat), jnp.float32),
            pltpu.VMEM((_C, 2 * f_feat), jnp.float32),
            pltpu.VMEM((_C, 2 * f_feat), jnp.float32),
            pltpu.SemaphoreType.DMA,
            pltpu.SemaphoreType.DMA,
            pltpu.SemaphoreType.DMA,
            pltpu.SemaphoreType.DMA,
        ],
    )
    def sc_kernel(x_hbm, idxs_hbm, out_hbm, xs_shared, idx_all, rows0, rows1,
                  out_v0, out_v1, sem_g0, sem_g1, sem_o0, sem_o1):
        sid = lax.axis_index("s")
        wid = sid * _NC + lax.axis_index("c")
        chunk0 = jnp.minimum(wid * chunks_per_worker,
                             n_chunks - chunks_per_worker)

        pltpu.async_copy(
            idxs_hbm.at[pl.ds(chunk0 * cw, chunks_per_worker * cw)],
            idx_all, sem_g1)

        @pl.when(sid < _NS - 1)
        def _():
            pltpu.sync_copy(x_hbm.at[pl.ds(sid * rpt, rpt)],
                            xs_shared.at[pl.ds(sid * rpt, rpt)])

        @pl.when(sid == _NS - 1)
        def _():
            pltpu.sync_copy(x_hbm.at[pl.ds((_NS - 1) * rpt, tail)],
                            xs_shared.at[pl.ds((_NS - 1) * rpt, tail)])

        pltpu.make_async_copy(
            idxs_hbm.at[pl.ds(chunk0 * cw, chunks_per_worker * cw)],
            idx_all, sem_g1).wait()
        plsc.subcore_barrier()

        def start_gather(ci, rows, sem):
            pltpu.async_copy(xs_shared.at[idx_all.at[pl.ds(ci * cw, cw)]],
                             rows, sem)

        def compute_chunk(ci, rows, out_v, sem_o):
            zero = jnp.zeros((_L,), jnp.float32)
            ninf = jnp.full((_L,), -jnp.inf, jnp.float32)
            for v in range(_C):
                r0 = v * k_nb

                def k_body(kq, acc, r0=r0):
                    sums, maxs = acc
                    for dk in range(4):
                        r = r0 + kq * 4 + dk
                        vals = [rows[r, pl.ds(f * _L, _L)] for f in range(nf)]
                        sums = tuple(s + x for s, x in zip(sums, vals))
                        maxs = tuple(jnp.maximum(m, x)
                                     for m, x in zip(maxs, vals))
                    return sums, maxs

                sums, maxs = lax.fori_loop(
                    0, k_nb // 4, k_body, ((zero,) * nf, (ninf,) * nf))
                for f in range(nf):
                    out_v[v, pl.ds(f * _L, _L)] = sums[f] * inv_k
                    out_v[v, pl.ds(f_feat + f * _L, _L)] = maxs[f]
            base = (chunk0 + ci) * _C
            pltpu.async_copy(out_v, out_hbm.at[pl.ds(base, _C)], sem_o)

        start_gather(0, rows0, sem_g0)

        def pair_body(i, carry):
            ci0 = i * 2
            start_gather(ci0 + 1, rows1, sem_g1)
            pltpu.make_async_copy(
                xs_shared.at[idx_all.at[pl.ds(0, cw)]], rows0, sem_g0).wait()

            @pl.when(i > 0)
            def _():
                pltpu.make_async_copy(out_v0, out_hbm.at[pl.ds(0, _C)],
                                      sem_o0).wait()
            compute_chunk(ci0, rows0, out_v0, sem_o0)

            @pl.when(ci0 + 2 < chunks_per_worker)
            def _():
                start_gather(ci0 + 2, rows0, sem_g0)
            pltpu.make_async_copy(
                xs_shared.at[idx_all.at[pl.ds(0, cw)]], rows1, sem_g1).wait()

            @pl.when(i > 0)
            def _():
                pltpu.make_async_copy(out_v1, out_hbm.at[pl.ds(0, _C)],
                                      sem_o1).wait()
            compute_chunk(ci0 + 1, rows1, out_v1, sem_o1)
            return carry

        lax.fori_loop(0, chunks_per_worker // 2, pair_body, 0)

        pltpu.make_async_copy(out_v0, out_hbm.at[pl.ds(0, _C)], sem_o0).wait()
        pltpu.make_async_copy(out_v1, out_hbm.at[pl.ds(0, _C)], sem_o1).wait()

    return sc_kernel


def kernel(x, idxs):
    n, f_feat = x.shape
    k_nb = idxs.shape[1]
    assert k_nb % 4 == 0 and f_feat % _L == 0 and n % _C == 0
    n_chunks = n // _C
    chunks_per_worker = ((n_chunks + 2 * _NW - 1) // (2 * _NW)) * 2
    assert n_chunks >= chunks_per_worker

    sc_kernel = _make_sc_kernel(n, k_nb, f_feat, chunks_per_worker)
    return sc_kernel(x, idxs.reshape(-1))


# speedup vs baseline: 1.0290x; 1.0027x over previous
import functools

import jax
import jax.numpy as jnp
from jax import lax
from jax.experimental import pallas as pl
from jax.experimental.pallas import tpu as pltpu
from jax.experimental.pallas import tpu_sc as plsc

_NC = 2
_NS = 16
_NW = _NC * _NS
_C = 4
_L = 16


def _make_sc_kernel(n, k_nb, f_feat, chunks_per_worker):
    nf = f_feat // _L
    inv_k = 1.0 / float(k_nb)
    cw = _C * k_nb
    rpt = ((n + _NS * 8 - 1) // (_NS * 8)) * 8
    tail = n - (_NS - 1) * rpt
    assert 0 < tail <= rpt and tail % 8 == 0
    n_chunks = n // _C
    mesh = plsc.VectorSubcoreMesh(core_axis_name="c", subcore_axis_name="s")

    @functools.partial(
        pl.kernel,
        mesh=mesh,
        out_type=jax.ShapeDtypeStruct((n, 2 * f_feat), jnp.float32),
        scratch_types=[
            pltpu.VMEM_SHARED((n, f_feat), jnp.float32),
            pltpu.VMEM((chunks_per_worker * cw,), jnp.int32),
            pltpu.VMEM((cw, f_feat), jnp.float32),
            pltpu.VMEM((cw, f_feat), jnp.float32),
            pltpu.VMEM((_C, 2 * f_feat), jnp.float32),
            pltpu.VMEM((_C, 2 * f_feat), jnp.float32),
            pltpu.SemaphoreType.DMA,
            pltpu.SemaphoreType.DMA,
            pltpu.SemaphoreType.DMA,
            pltpu.SemaphoreType.DMA,
        ],
    )
    def sc_kernel(x_hbm, idxs_hbm, out_hbm, xs_shared, idx_all, rows0, rows1,
                  out_v0, out_v1, sem_g0, sem_g1, sem_o0, sem_o1):
        sid = lax.axis_index("s")
        wid = sid * _NC + lax.axis_index("c")
        chunk0 = jnp.minimum(wid * chunks_per_worker,
                             n_chunks - chunks_per_worker)

        pltpu.async_copy(
            idxs_hbm.at[pl.ds(chunk0 * cw, chunks_per_worker * cw)],
            idx_all, sem_g1)

        @pl.when(sid < _NS - 1)
        def _():
            pltpu.sync_copy(x_hbm.at[pl.ds(sid * rpt, rpt)],
                            xs_shared.at[pl.ds(sid * rpt, rpt)])

        @pl.when(sid == _NS - 1)
        def _():
            pltpu.sync_copy(x_hbm.at[pl.ds((_NS - 1) * rpt, tail)],
                            xs_shared.at[pl.ds((_NS - 1) * rpt, tail)])

        pltpu.make_async_copy(
            idxs_hbm.at[pl.ds(chunk0 * cw, chunks_per_worker * cw)],
            idx_all, sem_g1).wait()
        plsc.subcore_barrier()

        def start_gather(ci, rows, sem):
            pltpu.async_copy(xs_shared.at[idx_all.at[pl.ds(ci * cw, cw)]],
                             rows, sem)

        def compute_chunk(ci, rows, out_v, sem_o):
            zero = jnp.zeros((_L,), jnp.float32)
            ninf = jnp.full((_L,), -jnp.inf, jnp.float32)
            for v in range(_C):
                r0 = v * k_nb

                def k_body(kq, acc, r0=r0):
                    sums, maxs = acc
                    for dk in range(4):
                        r = r0 + kq * 4 + dk
                        vals = [rows[r, pl.ds(f * _L, _L)] for f in range(nf)]
                        sums = tuple(s + x for s, x in zip(sums, vals))
                        maxs = tuple(jnp.maximum(m, x)
                                     for m, x in zip(maxs, vals))
                    return sums, maxs

                sums, maxs = lax.fori_loop(
                    0, k_nb // 4, k_body, ((zero,) * nf, (ninf,) * nf))
                for f in range(nf):
                    out_v[v, pl.ds(f * _L, _L)] = sums[f] * inv_k
                    out_v[v, pl.ds(f_feat + f * _L, _L)] = maxs[f]
            base = (chunk0 + ci) * _C
            pltpu.async_copy(out_v, out_hbm.at[pl.ds(base, _C)], sem_o)

        start_gather(0, rows0, sem_g0)

        def pair_body(i, carry):
            ci0 = i * 2
            start_gather(ci0 + 1, rows1, sem_g1)
            pltpu.make_async_copy(
                xs_shared.at[idx_all.at[pl.ds(0, cw)]], rows0, sem_g0).wait()

            @pl.when(i > 0)
            def _():
                pltpu.make_async_copy(out_v0, out_hbm.at[pl.ds(0, _C)],
                                      sem_o0).wait()
            compute_chunk(ci0, rows0, out_v0, sem_o0)

            @pl.when(ci0 + 2 < chunks_per_worker)
            def _():
                start_gather(ci0 + 2, rows0, sem_g0)
            pltpu.make_async_copy(
                xs_shared.at[idx_all.at[pl.ds(0, cw)]], rows1, sem_g1).wait()

            @pl.when(i > 0)
            def _():
                pltpu.make_async_copy(out_v1, out_hbm.at[pl.ds(0, _C)],
                                      sem_o1).wait()
            compute_chunk(ci0 + 1, rows1, out_v1, sem_o1)
            return carry

        lax.fori_loop(0, chunks_per_worker // 2, pair_body, 0)

        if chunks_per_worker % 2:
            last = chunks_per_worker - 1
            pltpu.make_async_copy(
                xs_shared.at[idx_all.at[pl.ds(0, cw)]], rows0, sem_g0).wait()
            pltpu.make_async_copy(out_v0, out_hbm.at[pl.ds(0, _C)],
                                  sem_o0).wait()
            compute_chunk(last, rows0, out_v0, sem_o0)

        pltpu.make_async_copy(out_v0, out_hbm.at[pl.ds(0, _C)], sem_o0).wait()
        pltpu.make_async_copy(out_v1, out_hbm.at[pl.ds(0, _C)], sem_o1).wait()

    return sc_kernel


def kernel(x, idxs):
    n, f_feat = x.shape
    k_nb = idxs.shape[1]
    assert k_nb % 4 == 0 and f_feat % _L == 0 and n % _C == 0
    n_chunks = n // _C
    chunks_per_worker = (n_chunks + _NW - 1) // _NW
    assert n_chunks >= chunks_per_worker and chunks_per_worker >= 2

    sc_kernel = _make_sc_kernel(n, k_nb, f_feat, chunks_per_worker)
    return sc_kernel(x, idxs.reshape(-1))
